# baseline (speedup 1.0000x reference)
import functools

import jax
import jax.numpy as jnp
from jax import lax
from jax.experimental import pallas as pl
from jax.experimental.pallas import tpu as pltpu
from jax.experimental.pallas import tpu_sc as plsc

_L = 16
_NW = 32
_K = 8
_QCHUNK = 128


def _sort16(x):
    k, _ = plsc.sort_key_val(x, x)
    return k


def _merge16(a, b):
    lo = jnp.minimum(a, lax.rev(b, (0,)))
    return _sort16(lo)


def _sc_knn_body(per_w, qchunk, wpb, keys_hbm, qb_hbm, out_hbm,
                 kx_v, ky_v, kz_v, qc_v, res_v):
    wid = lax.axis_index("s") * 2 + lax.axis_index("c")
    b = wid // wpb
    q0 = wid * per_w
    lane = jax.lax.iota(jnp.int32, _L)
    inf = jnp.float32(jnp.inf)
    M = kx_v.shape[0]

    pltpu.sync_copy(keys_hbm.at[pl.ds(pl.multiple_of((b * 3 + 0) * M, M), M)], kx_v)
    pltpu.sync_copy(keys_hbm.at[pl.ds(pl.multiple_of((b * 3 + 1) * M, M), M)], ky_v)
    pltpu.sync_copy(keys_hbm.at[pl.ds(pl.multiple_of((b * 3 + 2) * M, M), M)], kz_v)

    nchunk = M // _L

    def chunk_loop(qx, qy, qz, body, init):
        def step(c, carry):
            o = pl.multiple_of(c * _L, _L)
            kxc = kx_v[pl.ds(o, _L)]
            kyc = ky_v[pl.ds(o, _L)]
            kzc = kz_v[pl.ds(o, _L)]
            dx = qx - kxc
            dy = qy - kyc
            dz = qz - kzc
            d2 = dx * dx + dy * dy + dz * dz
            return body(d2, kxc, kyc, kzc, carry)
        return plsc.parallel_loop(0, nchunk, 1, unroll=4, carry=init)(step)

    def query_body(qi, _):
        qcl = qi % qchunk
        qo = pl.multiple_of(qcl * 3 * _L, _L)
        qx = qc_v[pl.ds(qo, _L)]
        qy = qc_v[pl.ds(qo + _L, _L)]
        qz = qc_v[pl.ds(qo + 2 * _L, _L)]

        def insert(d2, kxc, kyc, kzc, rs):
            t = d2
            out = []
            for r in rs:
                out.append(jnp.minimum(r, t))
                t = jnp.maximum(r, t)
            return tuple(out)

        rs = chunk_loop(qx, qy, qz, insert,
                        tuple(jnp.full((_L,), inf) for _ in range(_K)))

        s = [_sort16(r) for r in rs]
        m01 = _merge16(s[0], s[1])
        m23 = _merge16(s[2], s[3])
        m45 = _merge16(s[4], s[5])
        m67 = _merge16(s[6], s[7])
        mA = _merge16(m01, m23)
        mB = _merge16(m45, m67)
        top16 = _merge16(mA, mB)
        t8 = jnp.max(jnp.where(lane < _K, top16, -inf))
        t8v = jnp.full((_L,), t8)

        def accum(d2, kxc, kyc, kzc, carry):
            sx, sy, sz, sc = carry
            m = d2 <= t8v
            zero = jnp.zeros((_L,), jnp.float32)
            sx = sx + jnp.where(m, kxc, zero)
            sy = sy + jnp.where(m, kyc, zero)
            sz = sz + jnp.where(m, kzc, zero)
            sc = sc + jnp.where(m, jnp.ones((_L,), jnp.float32), zero)
            return sx, sy, sz, sc

        zero = jnp.zeros((_L,), jnp.float32)
        sx, sy, sz, sc = chunk_loop(qx, qy, qz, accum, (zero, zero, zero, zero))
        sxs = jnp.sum(sx)
        sys_ = jnp.sum(sy)
        szs = jnp.sum(sz)
        scs = jnp.sum(sc)
        res = jnp.where(lane == 0, sxs,
              jnp.where(lane == 1, sys_,
              jnp.where(lane == 2, szs,
              jnp.where(lane == 3, scs, 0.0)))).astype(jnp.float32)
        res_v[pl.ds(pl.multiple_of(qcl * _L, _L), _L)] = res
        return 0

    def qchunk_body(ch, _):
        qhbm0 = pl.multiple_of((q0 + ch * qchunk) * 3 * _L, _L)
        pltpu.sync_copy(qb_hbm.at[pl.ds(qhbm0, qchunk * 3 * _L)], qc_v)
        lax.fori_loop(0, qchunk, query_body, 0)
        ohbm0 = pl.multiple_of((q0 + ch * qchunk) * _L, _L)
        pltpu.sync_copy(res_v, out_hbm.at[pl.ds(ohbm0, qchunk * _L)])
        return 0

    lax.fori_loop(0, per_w // qchunk, qchunk_body, 0)


def _sc_knn(partial, predicted):
    B, M, _ = partial.shape
    _, N, _ = predicted.shape
    BN = B * N
    per_w = BN // _NW

    keys_flat = partial.transpose(0, 2, 1).reshape(B * 3 * M)
    qb = jnp.broadcast_to(predicted.reshape(BN, 3, 1),
                          (BN, 3, _L)).reshape(BN * 3 * _L)

    qchunk = min(_QCHUNK, per_w)
    wpb = _NW // B
    mesh = plsc.VectorSubcoreMesh(core_axis_name="c", subcore_axis_name="s")
    knn = pl.kernel(
        functools.partial(_sc_knn_body, per_w, qchunk, wpb),
        mesh=mesh,
        compiler_params=pltpu.CompilerParams(needs_layout_passes=False),
        out_type=jax.ShapeDtypeStruct((BN * _L,), jnp.float32),
        scratch_types=[
            pltpu.VMEM((M,), jnp.float32),
            pltpu.VMEM((M,), jnp.float32),
            pltpu.VMEM((M,), jnp.float32),
            pltpu.VMEM((qchunk * 3 * _L,), jnp.float32),
            pltpu.VMEM((qchunk * _L,), jnp.float32),
        ],
    )
    res = knn(keys_flat, qb)
    flat = res.reshape(BN, _L)
    nmean = flat[:, :3] / flat[:, 3:4]
    return nmean.reshape(B, N, 3)


def _mlp_body(pred_ref, nm_ref, w1t_ref, b1_ref, w2t_ref, b2_ref,
              wd0_ref, wd1_ref, bd_ref, wct_ref, bc_ref, o0_ref, o1_ref):
    pred = pred_ref[0]
    nmean = nm_ref[0]
    combined = jnp.concatenate([pred, nmean], axis=1)
    h = jax.nn.relu(
        lax.dot_general(combined, w1t_ref[...], (((1,), (0,)), ((), ())),
                        preferred_element_type=jnp.float32) + b1_ref[...])
    seed_feat = lax.dot_general(h, w2t_ref[...], (((1,), (0,)), ((), ())),
                                preferred_element_type=jnp.float32) + b2_ref[...]

    def head(wd_ref):
        hj = jax.nn.relu(
            lax.dot_general(seed_feat, wd_ref[...], (((1,), (0,)), ((), ())),
                            preferred_element_type=jnp.float32) + bd_ref[...])
        return lax.dot_general(hj, wct_ref[...], (((1,), (0,)), ((), ())),
                               preferred_element_type=jnp.float32) + bc_ref[...]

    o0_ref[0] = pred + head(wd0_ref)
    o1_ref[0] = pred + head(wd1_ref)


@jax.jit
def kernel(partial, predicted, W1, b1, W2, b2, Wd, bd, Wc, bc):
    B, M, _ = partial.shape
    _, N, _ = predicted.shape
    H = W1.shape[0]
    qb = 512

    nmean = _sc_knn(partial, predicted)

    w1t, w2t, wct = W1.T, W2.T, Wc.T
    wd0, wd1 = Wd[:, :, 0], Wd[:, :, 1]
    b1r, b2r = b1.reshape(1, H), b2.reshape(1, H)
    bdr, bcr = bd.reshape(1, H), bc.reshape(1, 3)

    full = lambda shape: pl.BlockSpec(shape, lambda i, j: (0,) * len(shape))
    o0, o1 = pl.pallas_call(
        _mlp_body,
        grid=(B, N // qb),
        in_specs=[
            pl.BlockSpec((1, qb, 3), lambda i, j: (i, j, 0)),
            pl.BlockSpec((1, qb, 3), lambda i, j: (i, j, 0)),
            full((6, H)), full((1, H)), full((H, H)), full((1, H)),
            full((H, H)), full((H, H)), full((1, H)),
            full((H, 3)), full((1, 3)),
        ],
        out_specs=[
            pl.BlockSpec((1, qb, 3), lambda i, j: (i, j, 0)),
            pl.BlockSpec((1, qb, 3), lambda i, j: (i, j, 0)),
        ],
        out_shape=[
            jax.ShapeDtypeStruct((B, N, 3), jnp.float32),
            jax.ShapeDtypeStruct((B, N, 3), jnp.float32),
        ],
        compiler_params=pltpu.CompilerParams(
            dimension_semantics=("parallel", "parallel")),
    )(predicted, nmean, w1t, b1r, w2t, b2r, wd0, wd1, bdr, wct, bcr)

    out = jnp.stack([o0, o1], axis=2)
    return out.reshape(B, N * 2, 3)


# speedup vs baseline: 3.2326x; 3.2326x over previous
import functools

import jax
import jax.numpy as jnp
from jax import lax
from jax.experimental import pallas as pl
from jax.experimental.pallas import tpu as pltpu
from jax.experimental.pallas import tpu_sc as plsc

_K = 8
_QB = 128
_L = 16
_NW = 32
_QCHUNK = 32
_NSC = 1152


def _mlp_block(pred, nmean, w1t_ref, b1_ref, w2t_ref, b2_ref,
               wd0_ref, wd1_ref, bd_ref, wct_ref, bc_ref):
    combined = jnp.concatenate([pred, nmean], axis=1)
    h = jax.nn.relu(
        lax.dot_general(combined, w1t_ref[...], (((1,), (0,)), ((), ())),
                        preferred_element_type=jnp.float32) + b1_ref[...])
    seed_feat = lax.dot_general(h, w2t_ref[...], (((1,), (0,)), ((), ())),
                                preferred_element_type=jnp.float32) + b2_ref[...]

    def head(wd_ref):
        hj = jax.nn.relu(
            lax.dot_general(seed_feat, wd_ref[...], (((1,), (0,)), ((), ())),
                            preferred_element_type=jnp.float32) + bd_ref[...])
        return lax.dot_general(hj, wct_ref[...], (((1,), (0,)), ((), ())),
                               preferred_element_type=jnp.float32) + bc_ref[...]

    return pred + head(wd0_ref), pred + head(wd1_ref)


def _fused_body(part_ref, partt_ref, pred_ref, w1t_ref, b1_ref, w2t_ref,
                b2_ref, wd0_ref, wd1_ref, bd_ref, wct_ref, bc_ref,
                o0_ref, o1_ref):
    part = part_ref[0]
    partt = partt_ref[0]
    pred = pred_ref[0]

    qq = jnp.sum(pred * pred, axis=1, keepdims=True)
    kk = jnp.sum(partt * partt, axis=0, keepdims=True)
    qk = lax.dot_general(pred, partt, (((1,), (0,)), ((), ())),
                         preferred_element_type=jnp.float32)
    d2 = qq + kk - 2.0 * qk

    def round_fn(_, t):
        return jnp.min(jnp.where(d2 > t, d2, jnp.inf), axis=1, keepdims=True)

    t = lax.fori_loop(0, _K, round_fn,
                      jnp.full((d2.shape[0], 1), -jnp.inf, jnp.float32))
    w = (d2 <= t).astype(jnp.float32)
    nsum = lax.dot_general(w, part, (((1,), (0,)), ((), ())),
                           preferred_element_type=jnp.float32)
    nmean = nsum[:, :3] / nsum[:, 3:4]

    o0, o1 = _mlp_block(pred, nmean, w1t_ref, b1_ref, w2t_ref, b2_ref,
                        wd0_ref, wd1_ref, bd_ref, wct_ref, bc_ref)
    o0_ref[0] = o0
    o1_ref[0] = o1


def _mlp_body(pred_ref, nm_ref, w1t_ref, b1_ref, w2t_ref, b2_ref,
              wd0_ref, wd1_ref, bd_ref, wct_ref, bc_ref, o0_ref, o1_ref):
    o0, o1 = _mlp_block(pred_ref[0], nm_ref[0], w1t_ref, b1_ref, w2t_ref,
                        b2_ref, wd0_ref, wd1_ref, bd_ref, wct_ref, bc_ref)
    o0_ref[0] = o0
    o1_ref[0] = o1


def _sort16(x):
    k, _ = plsc.sort_key_val(x, x)
    return k


def _merge16(a, b):
    lo = jnp.minimum(a, lax.rev(b, (0,)))
    return _sort16(lo)


def _sc_knn_body(per_w, qchunk, wpb, keys_hbm, qb_hbm, out_hbm,
                 kx_v, ky_v, kz_v, qc_v, res_v):
    wid = lax.axis_index("s") * 2 + lax.axis_index("c")
    b = wid // wpb
    q0 = wid * per_w
    lane = jax.lax.iota(jnp.int32, _L)
    inf = jnp.float32(jnp.inf)
    M = kx_v.shape[0]

    pltpu.sync_copy(keys_hbm.at[pl.ds(pl.multiple_of((b * 3 + 0) * M, M), M)], kx_v)
    pltpu.sync_copy(keys_hbm.at[pl.ds(pl.multiple_of((b * 3 + 1) * M, M), M)], ky_v)
    pltpu.sync_copy(keys_hbm.at[pl.ds(pl.multiple_of((b * 3 + 2) * M, M), M)], kz_v)

    nchunk = M // _L

    def chunk_loop(qx, qy, qz, body, init):
        def step(c, carry):
            o = pl.multiple_of(c * _L, _L)
            kxc = kx_v[pl.ds(o, _L)]
            kyc = ky_v[pl.ds(o, _L)]
            kzc = kz_v[pl.ds(o, _L)]
            dx = qx - kxc
            dy = qy - kyc
            dz = qz - kzc
            d2 = dx * dx + dy * dy + dz * dz
            return body(d2, kxc, kyc, kzc, carry)
        return plsc.parallel_loop(0, nchunk, 1, unroll=4, carry=init)(step)

    def query_body(qi, _):
        qcl = qi % qchunk
        qo = pl.multiple_of(qcl * 3 * _L, _L)
        qx = qc_v[pl.ds(qo, _L)]
        qy = qc_v[pl.ds(qo + _L, _L)]
        qz = qc_v[pl.ds(qo + 2 * _L, _L)]

        def insert(d2, kxc, kyc, kzc, rs):
            t = d2
            out = []
            for r in rs:
                out.append(jnp.minimum(r, t))
                t = jnp.maximum(r, t)
            return tuple(out)

        rs = chunk_loop(qx, qy, qz, insert,
                        tuple(jnp.full((_L,), inf) for _ in range(_K)))

        s = [_sort16(r) for r in rs]
        m01 = _merge16(s[0], s[1])
        m23 = _merge16(s[2], s[3])
        m45 = _merge16(s[4], s[5])
        m67 = _merge16(s[6], s[7])
        top16 = _merge16(_merge16(m01, m23), _merge16(m45, m67))
        t8 = jnp.max(jnp.where(lane < _K, top16, -inf))
        t8v = jnp.full((_L,), t8)

        def accum(d2, kxc, kyc, kzc, carry):
            sx, sy, sz, sc = carry
            m = d2 <= t8v
            zero = jnp.zeros((_L,), jnp.float32)
            sx = sx + jnp.where(m, kxc, zero)
            sy = sy + jnp.where(m, kyc, zero)
            sz = sz + jnp.where(m, kzc, zero)
            sc = sc + jnp.where(m, jnp.ones((_L,), jnp.float32), zero)
            return sx, sy, sz, sc

        zero = jnp.zeros((_L,), jnp.float32)
        sx, sy, sz, sc = chunk_loop(qx, qy, qz, accum, (zero, zero, zero, zero))
        sxs = jnp.sum(sx)
        sys_ = jnp.sum(sy)
        szs = jnp.sum(sz)
        scs = jnp.sum(sc)
        res = jnp.where(lane == 0, sxs,
              jnp.where(lane == 1, sys_,
              jnp.where(lane == 2, szs,
              jnp.where(lane == 3, scs, 0.0)))).astype(jnp.float32)
        res_v[pl.ds(pl.multiple_of(qcl * _L, _L), _L)] = res
        return 0

    def qchunk_body(ch, _):
        qhbm0 = pl.multiple_of((q0 + ch * qchunk) * 3 * _L, _L)
        pltpu.sync_copy(qb_hbm.at[pl.ds(qhbm0, qchunk * 3 * _L)], qc_v)
        lax.fori_loop(0, qchunk, query_body, 0)
        ohbm0 = pl.multiple_of((q0 + ch * qchunk) * _L, _L)
        pltpu.sync_copy(res_v, out_hbm.at[pl.ds(ohbm0, qchunk * _L)])
        return 0

    lax.fori_loop(0, per_w // qchunk, qchunk_body, 0)


def _sc_knn(partial, pred_slice):
    B, M, _ = partial.shape
    _, NS, _ = pred_slice.shape
    BN = B * NS
    per_w = BN // _NW

    keys_flat = partial.transpose(0, 2, 1).reshape(B * 3 * M)
    qbv = jnp.broadcast_to(pred_slice.reshape(BN, 3, 1),
                           (BN, 3, _L)).reshape(BN * 3 * _L)

    qchunk = min(_QCHUNK, per_w)
    wpb = _NW // B
    mesh = plsc.VectorSubcoreMesh(core_axis_name="c", subcore_axis_name="s")
    knn = pl.kernel(
        functools.partial(_sc_knn_body, per_w, qchunk, wpb),
        mesh=mesh,
        compiler_params=pltpu.CompilerParams(needs_layout_passes=False),
        out_type=jax.ShapeDtypeStruct((BN * _L,), jnp.float32),
        scratch_types=[
            pltpu.VMEM((M,), jnp.float32),
            pltpu.VMEM((M,), jnp.float32),
            pltpu.VMEM((M,), jnp.float32),
            pltpu.VMEM((qchunk * 3 * _L,), jnp.float32),
            pltpu.VMEM((qchunk * _L,), jnp.float32),
        ],
    )
    res = knn(keys_flat, qbv)
    flat = res.reshape(BN, _L)
    nmean = flat[:, :3] / flat[:, 3:4]
    return nmean.reshape(B, NS, 3)


@jax.jit
def kernel(partial, predicted, W1, b1, W2, b2, Wd, bd, Wc, bc):
    B, M, _ = partial.shape
    _, N, _ = predicted.shape
    H = W1.shape[0]

    nsc = _NSC if (N > _NSC and ((N - _NSC) % _QB == 0)
                   and (B * _NSC) % (_NW * _QCHUNK) == 0) else 0
    nt = N - nsc

    w1t, w2t, wct = W1.T, W2.T, Wc.T
    wd0, wd1 = Wd[:, :, 0], Wd[:, :, 1]
    b1r, b2r = b1.reshape(1, H), b2.reshape(1, H)
    bdr, bcr = bd.reshape(1, H), bc.reshape(1, 3)
    part4 = jnp.concatenate([partial, jnp.ones((B, M, 1), jnp.float32)], axis=2)
    partt = partial.transpose(0, 2, 1)
    weights = (w1t, b1r, w2t, b2r, wd0, wd1, bdr, wct, bcr)

    full = lambda shape: pl.BlockSpec(shape, lambda i, j: (0,) * len(shape))
    wspecs = [full((6, H)), full((1, H)), full((H, H)), full((1, H)),
              full((H, H)), full((H, H)), full((1, H)),
              full((H, 3)), full((1, 3))]

    qb = min(_QB, nt if nsc else N)
    pred_tc = predicted[:, :nt] if nsc else predicted

    if nsc:
        nmean_sc = _sc_knn(partial, predicted[:, nt:])

    o0t, o1t = pl.pallas_call(
        _fused_body,
        grid=(B, nt // qb),
        in_specs=[
            pl.BlockSpec((1, M, 4), lambda i, j: (i, 0, 0)),
            pl.BlockSpec((1, 3, M), lambda i, j: (i, 0, 0)),
            pl.BlockSpec((1, qb, 3), lambda i, j: (i, j, 0)),
        ] + wspecs,
        out_specs=[
            pl.BlockSpec((1, qb, 3), lambda i, j: (i, j, 0)),
            pl.BlockSpec((1, qb, 3), lambda i, j: (i, j, 0)),
        ],
        out_shape=[
            jax.ShapeDtypeStruct((B, nt, 3), jnp.float32),
            jax.ShapeDtypeStruct((B, nt, 3), jnp.float32),
        ],
        compiler_params=pltpu.CompilerParams(
            dimension_semantics=("parallel", "parallel")),
    )(part4, partt, pred_tc, *weights)

    if nsc:
        qbs = next(q for q in (512, 384, 256, 128, nsc) if nsc % q == 0)
        o0s, o1s = pl.pallas_call(
            _mlp_body,
            grid=(B, nsc // qbs),
            in_specs=[
                pl.BlockSpec((1, qbs, 3), lambda i, j: (i, j, 0)),
                pl.BlockSpec((1, qbs, 3), lambda i, j: (i, j, 0)),
            ] + wspecs,
            out_specs=[
                pl.BlockSpec((1, qbs, 3), lambda i, j: (i, j, 0)),
                pl.BlockSpec((1, qbs, 3), lambda i, j: (i, j, 0)),
            ],
            out_shape=[
                jax.ShapeDtypeStruct((B, nsc, 3), jnp.float32),
                jax.ShapeDtypeStruct((B, nsc, 3), jnp.float32),
            ],
            compiler_params=pltpu.CompilerParams(
                dimension_semantics=("parallel", "parallel")),
        )(predicted[:, nt:], nmean_sc, *weights)
        o0 = jnp.concatenate([o0t, o0s], axis=1)
        o1 = jnp.concatenate([o1t, o1s], axis=1)
    else:
        o0, o1 = o0t, o1t

    out = jnp.stack([o0, o1], axis=2)
    return out.reshape(B, N * 2, 3)


# speedup vs baseline: 3.4558x; 1.0690x over previous
import functools

import jax
import jax.numpy as jnp
from jax import lax
from jax.experimental import pallas as pl
from jax.experimental.pallas import tpu as pltpu
from jax.experimental.pallas import tpu_sc as plsc

_K = 8
_QB = 128
_L = 16
_NW = 32
_QCHUNK = 32
_NSC = 1152


def _mlp_block(pred, nmean, w1t_ref, b1_ref, w2t_ref, b2_ref,
               wd0_ref, wd1_ref, bd_ref, wct_ref, bc_ref):
    combined = jnp.concatenate([pred, nmean], axis=1)
    h = jax.nn.relu(
        lax.dot_general(combined, w1t_ref[...], (((1,), (0,)), ((), ())),
                        preferred_element_type=jnp.float32) + b1_ref[...])
    seed_feat = lax.dot_general(h, w2t_ref[...], (((1,), (0,)), ((), ())),
                                preferred_element_type=jnp.float32) + b2_ref[...]

    def head(wd_ref):
        hj = jax.nn.relu(
            lax.dot_general(seed_feat, wd_ref[...], (((1,), (0,)), ((), ())),
                            preferred_element_type=jnp.float32) + bd_ref[...])
        return lax.dot_general(hj, wct_ref[...], (((1,), (0,)), ((), ())),
                               preferred_element_type=jnp.float32) + bc_ref[...]

    return pred + head(wd0_ref), pred + head(wd1_ref)


def _fused_body(part_ref, partt_ref, pred_ref, w1t_ref, b1_ref, w2t_ref,
                b2_ref, wd0_ref, wd1_ref, bd_ref, wct_ref, bc_ref,
                o0_ref, o1_ref):
    part = part_ref[0]
    partt = partt_ref[0]
    pred = pred_ref[0]

    qq = jnp.sum(pred * pred, axis=1, keepdims=True)
    kk = jnp.sum(partt * partt, axis=0, keepdims=True)
    qk = lax.dot_general(pred, partt, (((1,), (0,)), ((), ())),
                         preferred_element_type=jnp.float32)
    d2 = qq + kk - 2.0 * qk

    t = jnp.min(d2, axis=1, keepdims=True)
    for _ in range(_K - 1):
        t = jnp.min(jnp.where(d2 > t, d2, jnp.inf), axis=1, keepdims=True)
    w = (d2 <= t).astype(jnp.float32)
    nsum = lax.dot_general(w, part, (((1,), (0,)), ((), ())),
                           preferred_element_type=jnp.float32)
    nmean = nsum[:, :3] / nsum[:, 3:4]

    o0, o1 = _mlp_block(pred, nmean, w1t_ref, b1_ref, w2t_ref, b2_ref,
                        wd0_ref, wd1_ref, bd_ref, wct_ref, bc_ref)
    o0_ref[0] = o0
    o1_ref[0] = o1


def _mlp_body(pred_ref, nm_ref, w1t_ref, b1_ref, w2t_ref, b2_ref,
              wd0_ref, wd1_ref, bd_ref, wct_ref, bc_ref, o0_ref, o1_ref):
    o0, o1 = _mlp_block(pred_ref[0], nm_ref[0], w1t_ref, b1_ref, w2t_ref,
                        b2_ref, wd0_ref, wd1_ref, bd_ref, wct_ref, bc_ref)
    o0_ref[0] = o0
    o1_ref[0] = o1


def _sort16(x):
    k, _ = plsc.sort_key_val(x, x)
    return k


def _merge16(a, b):
    lo = jnp.minimum(a, lax.rev(b, (0,)))
    return _sort16(lo)


def _sc_knn_body(per_w, qchunk, wpb, keys_hbm, qb_hbm, out_hbm,
                 kx_v, ky_v, kz_v, qc_v, res_v):
    wid = lax.axis_index("s") * 2 + lax.axis_index("c")
    b = wid // wpb
    q0 = wid * per_w
    lane = jax.lax.iota(jnp.int32, _L)
    inf = jnp.float32(jnp.inf)
    M = kx_v.shape[0]

    pltpu.sync_copy(keys_hbm.at[pl.ds(pl.multiple_of((b * 3 + 0) * M, M), M)], kx_v)
    pltpu.sync_copy(keys_hbm.at[pl.ds(pl.multiple_of((b * 3 + 1) * M, M), M)], ky_v)
    pltpu.sync_copy(keys_hbm.at[pl.ds(pl.multiple_of((b * 3 + 2) * M, M), M)], kz_v)

    nchunk = M // _L

    def chunk_loop(qx, qy, qz, body, init):
        def step(c, carry):
            o = pl.multiple_of(c * _L, _L)
            kxc = kx_v[pl.ds(o, _L)]
            kyc = ky_v[pl.ds(o, _L)]
            kzc = kz_v[pl.ds(o, _L)]
            dx = qx - kxc
            dy = qy - kyc
            dz = qz - kzc
            d2 = dx * dx + dy * dy + dz * dz
            return body(d2, kxc, kyc, kzc, carry)
        return plsc.parallel_loop(0, nchunk, 1, unroll=4, carry=init)(step)

    def query_body(qi, _):
        qcl = qi % qchunk
        qo = pl.multiple_of(qcl * 3 * _L, _L)
        qx = qc_v[pl.ds(qo, _L)]
        qy = qc_v[pl.ds(qo + _L, _L)]
        qz = qc_v[pl.ds(qo + 2 * _L, _L)]

        def insert(d2, kxc, kyc, kzc, rs):
            t = d2
            out = []
            for r in rs:
                out.append(jnp.minimum(r, t))
                t = jnp.maximum(r, t)
            return tuple(out)

        rs = chunk_loop(qx, qy, qz, insert,
                        tuple(jnp.full((_L,), inf) for _ in range(_K)))

        s = [_sort16(r) for r in rs]
        m01 = _merge16(s[0], s[1])
        m23 = _merge16(s[2], s[3])
        m45 = _merge16(s[4], s[5])
        m67 = _merge16(s[6], s[7])
        top16 = _merge16(_merge16(m01, m23), _merge16(m45, m67))
        t8 = jnp.max(jnp.where(lane < _K, top16, -inf))
        t8v = jnp.full((_L,), t8)

        def accum(d2, kxc, kyc, kzc, carry):
            sx, sy, sz, sc = carry
            m = d2 <= t8v
            zero = jnp.zeros((_L,), jnp.float32)
            sx = sx + jnp.where(m, kxc, zero)
            sy = sy + jnp.where(m, kyc, zero)
            sz = sz + jnp.where(m, kzc, zero)
            sc = sc + jnp.where(m, jnp.ones((_L,), jnp.float32), zero)
            return sx, sy, sz, sc

        zero = jnp.zeros((_L,), jnp.float32)
        sx, sy, sz, sc = chunk_loop(qx, qy, qz, accum, (zero, zero, zero, zero))
        sxs = jnp.sum(sx)
        sys_ = jnp.sum(sy)
        szs = jnp.sum(sz)
        scs = jnp.sum(sc)
        res = jnp.where(lane == 0, sxs,
              jnp.where(lane == 1, sys_,
              jnp.where(lane == 2, szs,
              jnp.where(lane == 3, scs, 0.0)))).astype(jnp.float32)
        res_v[pl.ds(pl.multiple_of(qcl * _L, _L), _L)] = res
        return 0

    def qchunk_body(ch, _):
        qhbm0 = pl.multiple_of((q0 + ch * qchunk) * 3 * _L, _L)
        pltpu.sync_copy(qb_hbm.at[pl.ds(qhbm0, qchunk * 3 * _L)], qc_v)
        lax.fori_loop(0, qchunk, query_body, 0)
        ohbm0 = pl.multiple_of((q0 + ch * qchunk) * _L, _L)
        pltpu.sync_copy(res_v, out_hbm.at[pl.ds(ohbm0, qchunk * _L)])
        return 0

    lax.fori_loop(0, per_w // qchunk, qchunk_body, 0)


def _sc_knn(partial, pred_slice):
    B, M, _ = partial.shape
    _, NS, _ = pred_slice.shape
    BN = B * NS
    per_w = BN // _NW

    keys_flat = partial.transpose(0, 2, 1).reshape(B * 3 * M)
    qbv = jnp.broadcast_to(pred_slice.reshape(BN, 3, 1),
                           (BN, 3, _L)).reshape(BN * 3 * _L)

    qchunk = min(_QCHUNK, per_w)
    wpb = _NW // B
    mesh = plsc.VectorSubcoreMesh(core_axis_name="c", subcore_axis_name="s")
    knn = pl.kernel(
        functools.partial(_sc_knn_body, per_w, qchunk, wpb),
        mesh=mesh,
        compiler_params=pltpu.CompilerParams(needs_layout_passes=False),
        out_type=jax.ShapeDtypeStruct((BN * _L,), jnp.float32),
        scratch_types=[
            pltpu.VMEM((M,), jnp.float32),
            pltpu.VMEM((M,), jnp.float32),
            pltpu.VMEM((M,), jnp.float32),
            pltpu.VMEM((qchunk * 3 * _L,), jnp.float32),
            pltpu.VMEM((qchunk * _L,), jnp.float32),
        ],
    )
    res = knn(keys_flat, qbv)
    flat = res.reshape(BN, _L)
    nmean = flat[:, :3] / flat[:, 3:4]
    return nmean.reshape(B, NS, 3)


@jax.jit
def kernel(partial, predicted, W1, b1, W2, b2, Wd, bd, Wc, bc):
    B, M, _ = partial.shape
    _, N, _ = predicted.shape
    H = W1.shape[0]

    nsc = _NSC if (N > _NSC and ((N - _NSC) % _QB == 0)
                   and (B * _NSC) % (_NW * _QCHUNK) == 0) else 0
    nt = N - nsc

    w1t, w2t, wct = W1.T, W2.T, Wc.T
    wd0, wd1 = Wd[:, :, 0], Wd[:, :, 1]
    b1r, b2r = b1.reshape(1, H), b2.reshape(1, H)
    bdr, bcr = bd.reshape(1, H), bc.reshape(1, 3)
    part4 = jnp.concatenate([partial, jnp.ones((B, M, 1), jnp.float32)], axis=2)
    partt = partial.transpose(0, 2, 1)
    weights = (w1t, b1r, w2t, b2r, wd0, wd1, bdr, wct, bcr)

    full = lambda shape: pl.BlockSpec(shape, lambda i, j: (0,) * len(shape))
    wspecs = [full((6, H)), full((1, H)), full((H, H)), full((1, H)),
              full((H, H)), full((H, H)), full((1, H)),
              full((H, 3)), full((1, 3))]

    qb = min(_QB, nt if nsc else N)
    pred_tc = predicted[:, :nt] if nsc else predicted

    if nsc:
        nmean_sc = _sc_knn(partial, predicted[:, nt:])

    o0t, o1t = pl.pallas_call(
        _fused_body,
        grid=(B, nt // qb),
        in_specs=[
            pl.BlockSpec((1, M, 4), lambda i, j: (i, 0, 0)),
            pl.BlockSpec((1, 3, M), lambda i, j: (i, 0, 0)),
            pl.BlockSpec((1, qb, 3), lambda i, j: (i, j, 0)),
        ] + wspecs,
        out_specs=[
            pl.BlockSpec((1, qb, 3), lambda i, j: (i, j, 0)),
            pl.BlockSpec((1, qb, 3), lambda i, j: (i, j, 0)),
        ],
        out_shape=[
            jax.ShapeDtypeStruct((B, nt, 3), jnp.float32),
            jax.ShapeDtypeStruct((B, nt, 3), jnp.float32),
        ],
        compiler_params=pltpu.CompilerParams(
            dimension_semantics=("parallel", "parallel")),
    )(part4, partt, pred_tc, *weights)

    if nsc:
        qbs = next(q for q in (512, 384, 256, 128, nsc) if nsc % q == 0)
        o0s, o1s = pl.pallas_call(
            _mlp_body,
            grid=(B, nsc // qbs),
            in_specs=[
                pl.BlockSpec((1, qbs, 3), lambda i, j: (i, j, 0)),
                pl.BlockSpec((1, qbs, 3), lambda i, j: (i, j, 0)),
            ] + wspecs,
            out_specs=[
                pl.BlockSpec((1, qbs, 3), lambda i, j: (i, j, 0)),
                pl.BlockSpec((1, qbs, 3), lambda i, j: (i, j, 0)),
            ],
            out_shape=[
                jax.ShapeDtypeStruct((B, nsc, 3), jnp.float32),
                jax.ShapeDtypeStruct((B, nsc, 3), jnp.float32),
            ],
            compiler_params=pltpu.CompilerParams(
                dimension_semantics=("parallel", "parallel")),
        )(predicted[:, nt:], nmean_sc, *weights)
        o0 = jnp.concatenate([o0t, o0s], axis=1)
        o1 = jnp.concatenate([o1t, o1s], axis=1)
    else:
        o0, o1 = o0t, o1t

    out = jnp.stack([o0, o1], axis=2)
    return out.reshape(B, N * 2, 3)


# speedup vs baseline: 3.6600x; 1.0591x over previous
import functools

import jax
import jax.numpy as jnp
from jax import lax
from jax.experimental import pallas as pl
from jax.experimental.pallas import tpu as pltpu
from jax.experimental.pallas import tpu_sc as plsc

_K = 8
_QB = 128
_L = 16
_NW = 32
_QCHUNK = 32
_NSC = 1024


def _mlp_block(pred, nmean, w1t_ref, b1_ref, w2t_ref, b2_ref,
               wd0_ref, wd1_ref, bd_ref, wct_ref, bc_ref):
    combined = jnp.concatenate([pred, nmean], axis=1)
    h = jax.nn.relu(
        lax.dot_general(combined, w1t_ref[...], (((1,), (0,)), ((), ())),
                        preferred_element_type=jnp.float32) + b1_ref[...])
    seed_feat = lax.dot_general(h, w2t_ref[...], (((1,), (0,)), ((), ())),
                                preferred_element_type=jnp.float32) + b2_ref[...]

    def head(wd_ref):
        hj = jax.nn.relu(
            lax.dot_general(seed_feat, wd_ref[...], (((1,), (0,)), ((), ())),
                            preferred_element_type=jnp.float32) + bd_ref[...])
        return lax.dot_general(hj, wct_ref[...], (((1,), (0,)), ((), ())),
                               preferred_element_type=jnp.float32) + bc_ref[...]

    return pred + head(wd0_ref), pred + head(wd1_ref)


def _fused_body(part_ref, partt_ref, pred_ref, w1t_ref, b1_ref, w2t_ref,
                b2_ref, wd0_ref, wd1_ref, bd_ref, wct_ref, bc_ref,
                o0_ref, o1_ref):
    part = part_ref[0]
    partt = partt_ref[0]
    pred = pred_ref[0]

    qq = jnp.sum(pred * pred, axis=1, keepdims=True)
    kk = jnp.sum(partt * partt, axis=0, keepdims=True)
    qk = lax.dot_general(pred, partt, (((1,), (0,)), ((), ())),
                         preferred_element_type=jnp.float32)
    d2 = qq + kk - 2.0 * qk

    t = jnp.min(d2, axis=1, keepdims=True)
    for _ in range(_K - 1):
        t = jnp.min(jnp.where(d2 > t, d2, jnp.inf), axis=1, keepdims=True)
    w = (d2 <= t).astype(jnp.float32)
    nsum = lax.dot_general(w, part, (((1,), (0,)), ((), ())),
                           preferred_element_type=jnp.float32)
    cnt = jnp.sum(w, axis=1, keepdims=True)
    nmean = nsum / cnt

    o0, o1 = _mlp_block(pred, nmean, w1t_ref, b1_ref, w2t_ref, b2_ref,
                        wd0_ref, wd1_ref, bd_ref, wct_ref, bc_ref)
    o0_ref[0] = o0
    o1_ref[0] = o1


def _mlp_body(pred_ref, nm_ref, w1t_ref, b1_ref, w2t_ref, b2_ref,
              wd0_ref, wd1_ref, bd_ref, wct_ref, bc_ref, o0_ref, o1_ref):
    o0, o1 = _mlp_block(pred_ref[0], nm_ref[0], w1t_ref, b1_ref, w2t_ref,
                        b2_ref, wd0_ref, wd1_ref, bd_ref, wct_ref, bc_ref)
    o0_ref[0] = o0
    o1_ref[0] = o1


def _sort16(x):
    k, _ = plsc.sort_key_val(x, x)
    return k


def _merge16(a, b):
    lo = jnp.minimum(a, lax.rev(b, (0,)))
    return _sort16(lo)


def _sc_knn_body(per_w, qchunk, wpb, keys_hbm, qb_hbm, out_hbm,
                 kx_v, ky_v, kz_v, qc_v, res_v):
    wid = lax.axis_index("s") * 2 + lax.axis_index("c")
    b = wid // wpb
    q0 = wid * per_w
    lane = jax.lax.iota(jnp.int32, _L)
    inf = jnp.float32(jnp.inf)
    M = kx_v.shape[0]

    pltpu.sync_copy(keys_hbm.at[pl.ds(pl.multiple_of((b * 3 + 0) * M, M), M)], kx_v)
    pltpu.sync_copy(keys_hbm.at[pl.ds(pl.multiple_of((b * 3 + 1) * M, M), M)], ky_v)
    pltpu.sync_copy(keys_hbm.at[pl.ds(pl.multiple_of((b * 3 + 2) * M, M), M)], kz_v)

    nchunk = M // _L

    def chunk_loop(qx, qy, qz, body, init):
        def step(c, carry):
            o = pl.multiple_of(c * _L, _L)
            kxc = kx_v[pl.ds(o, _L)]
            kyc = ky_v[pl.ds(o, _L)]
            kzc = kz_v[pl.ds(o, _L)]
            dx = qx - kxc
            dy = qy - kyc
            dz = qz - kzc
            d2 = dx * dx + dy * dy + dz * dz
            return body(d2, kxc, kyc, kzc, carry)
        return plsc.parallel_loop(0, nchunk, 1, unroll=4, carry=init)(step)

    def query_body(qi, _):
        qcl = qi % qchunk
        qo = pl.multiple_of(qcl * 3 * _L, _L)
        qx = qc_v[pl.ds(qo, _L)]
        qy = qc_v[pl.ds(qo + _L, _L)]
        qz = qc_v[pl.ds(qo + 2 * _L, _L)]

        def insert(d2, kxc, kyc, kzc, rs):
            t = d2
            out = []
            for r in rs:
                out.append(jnp.minimum(r, t))
                t = jnp.maximum(r, t)
            return tuple(out)

        rs = chunk_loop(qx, qy, qz, insert,
                        tuple(jnp.full((_L,), inf) for _ in range(_K)))

        s = [_sort16(r) for r in rs]
        m01 = _merge16(s[0], s[1])
        m23 = _merge16(s[2], s[3])
        m45 = _merge16(s[4], s[5])
        m67 = _merge16(s[6], s[7])
        top16 = _merge16(_merge16(m01, m23), _merge16(m45, m67))
        t8 = jnp.max(jnp.where(lane < _K, top16, -inf))
        t8v = jnp.full((_L,), t8)

        def accum(d2, kxc, kyc, kzc, carry):
            sx, sy, sz, sc = carry
            m = d2 <= t8v
            zero = jnp.zeros((_L,), jnp.float32)
            sx = sx + jnp.where(m, kxc, zero)
            sy = sy + jnp.where(m, kyc, zero)
            sz = sz + jnp.where(m, kzc, zero)
            sc = sc + jnp.where(m, jnp.ones((_L,), jnp.float32), zero)
            return sx, sy, sz, sc

        zero = jnp.zeros((_L,), jnp.float32)
        sx, sy, sz, sc = chunk_loop(qx, qy, qz, accum, (zero, zero, zero, zero))
        sxs = jnp.sum(sx)
        sys_ = jnp.sum(sy)
        szs = jnp.sum(sz)
        scs = jnp.sum(sc)
        res = jnp.where(lane == 0, sxs,
              jnp.where(lane == 1, sys_,
              jnp.where(lane == 2, szs,
              jnp.where(lane == 3, scs, 0.0)))).astype(jnp.float32)
        res_v[pl.ds(pl.multiple_of(qcl * _L, _L), _L)] = res
        return 0

    def qchunk_body(ch, _):
        qhbm0 = pl.multiple_of((q0 + ch * qchunk) * 3 * _L, _L)
        pltpu.sync_copy(qb_hbm.at[pl.ds(qhbm0, qchunk * 3 * _L)], qc_v)
        lax.fori_loop(0, qchunk, query_body, 0)
        ohbm0 = pl.multiple_of((q0 + ch * qchunk) * _L, _L)
        pltpu.sync_copy(res_v, out_hbm.at[pl.ds(ohbm0, qchunk * _L)])
        return 0

    lax.fori_loop(0, per_w // qchunk, qchunk_body, 0)


def _sc_knn(partial, pred_slice):
    B, M, _ = partial.shape
    _, NS, _ = pred_slice.shape
    BN = B * NS
    per_w = BN // _NW

    keys_flat = partial.transpose(0, 2, 1).reshape(B * 3 * M)
    qbv = jnp.broadcast_to(pred_slice.reshape(BN, 3, 1),
                           (BN, 3, _L)).reshape(BN * 3 * _L)

    qchunk = min(_QCHUNK, per_w)
    wpb = _NW // B
    mesh = plsc.VectorSubcoreMesh(core_axis_name="c", subcore_axis_name="s")
    knn = pl.kernel(
        functools.partial(_sc_knn_body, per_w, qchunk, wpb),
        mesh=mesh,
        compiler_params=pltpu.CompilerParams(needs_layout_passes=False),
        out_type=jax.ShapeDtypeStruct((BN * _L,), jnp.float32),
        scratch_types=[
            pltpu.VMEM((M,), jnp.float32),
            pltpu.VMEM((M,), jnp.float32),
            pltpu.VMEM((M,), jnp.float32),
            pltpu.VMEM((qchunk * 3 * _L,), jnp.float32),
            pltpu.VMEM((qchunk * _L,), jnp.float32),
        ],
    )
    res = knn(keys_flat, qbv)
    flat = res.reshape(BN, _L)
    nmean = flat[:, :3] / flat[:, 3:4]
    return nmean.reshape(B, NS, 3)


@jax.jit
def kernel(partial, predicted, W1, b1, W2, b2, Wd, bd, Wc, bc):
    B, M, _ = partial.shape
    _, N, _ = predicted.shape
    H = W1.shape[0]

    nsc = _NSC if (N > _NSC and ((N - _NSC) % _QB == 0)
                   and (B * _NSC) % (_NW * _QCHUNK) == 0) else 0
    nt = N - nsc

    w1t, w2t, wct = W1.T, W2.T, Wc.T
    wd0, wd1 = Wd[:, :, 0], Wd[:, :, 1]
    b1r, b2r = b1.reshape(1, H), b2.reshape(1, H)
    bdr, bcr = bd.reshape(1, H), bc.reshape(1, 3)
    partt = partial.transpose(0, 2, 1)
    weights = (w1t, b1r, w2t, b2r, wd0, wd1, bdr, wct, bcr)

    full = lambda shape: pl.BlockSpec(shape, lambda i, j: (0,) * len(shape))
    wspecs = [full((6, H)), full((1, H)), full((H, H)), full((1, H)),
              full((H, H)), full((H, H)), full((1, H)),
              full((H, 3)), full((1, 3))]

    qb = min(_QB, nt if nsc else N)
    pred_tc = predicted[:, :nt] if nsc else predicted

    if nsc:
        nmean_sc = _sc_knn(partial, predicted[:, nt:])

    o0t, o1t = pl.pallas_call(
        _fused_body,
        grid=(B, nt // qb),
        in_specs=[
            pl.BlockSpec((1, M, 3), lambda i, j: (i, 0, 0)),
            pl.BlockSpec((1, 3, M), lambda i, j: (i, 0, 0)),
            pl.BlockSpec((1, qb, 3), lambda i, j: (i, j, 0)),
        ] + wspecs,
        out_specs=[
            pl.BlockSpec((1, qb, 3), lambda i, j: (i, j, 0)),
            pl.BlockSpec((1, qb, 3), lambda i, j: (i, j, 0)),
        ],
        out_shape=[
            jax.ShapeDtypeStruct((B, nt, 3), jnp.float32),
            jax.ShapeDtypeStruct((B, nt, 3), jnp.float32),
        ],
        compiler_params=pltpu.CompilerParams(
            dimension_semantics=("parallel", "parallel")),
    )(partial, partt, pred_tc, *weights)

    if nsc:
        qbs = next(q for q in (512, 384, 256, 128, nsc) if nsc % q == 0)
        o0s, o1s = pl.pallas_call(
            _mlp_body,
            grid=(B, nsc // qbs),
            in_specs=[
                pl.BlockSpec((1, qbs, 3), lambda i, j: (i, j, 0)),
                pl.BlockSpec((1, qbs, 3), lambda i, j: (i, j, 0)),
            ] + wspecs,
            out_specs=[
                pl.BlockSpec((1, qbs, 3), lambda i, j: (i, j, 0)),
                pl.BlockSpec((1, qbs, 3), lambda i, j: (i, j, 0)),
            ],
            out_shape=[
                jax.ShapeDtypeStruct((B, nsc, 3), jnp.float32),
                jax.ShapeDtypeStruct((B, nsc, 3), jnp.float32),
            ],
            compiler_params=pltpu.CompilerParams(
                dimension_semantics=("parallel", "parallel")),
        )(predicted[:, nt:], nmean_sc, *weights)
        o0 = jnp.concatenate([o0t, o0s], axis=1)
        o1 = jnp.concatenate([o1t, o1s], axis=1)
    else:
        o0, o1 = o0t, o1t

    out = jnp.stack([o0, o1], axis=2)
    return out.reshape(B, N * 2, 3)
